# baseline (speedup 1.0000x reference)
import functools

import jax
import jax.numpy as jnp
from jax import lax
from jax.experimental import pallas as pl
from jax.experimental.pallas import tpu as pltpu
from jax.experimental.pallas import tpu_sc as plsc

_NC = 2
_NS = 16
_NW = _NC * _NS


def _lookup_call(tables_t, users_t, num_fields, vocab, dim, batch):
    mesh = plsc.VectorSubcoreMesh(
        core_axis_name="c", subcore_axis_name="s",
        num_cores=_NC, num_subcores=_NS)

    @functools.partial(
        pl.kernel,
        mesh=mesh,
        out_type=jax.ShapeDtypeStruct((num_fields * dim, batch), jnp.float32),
        scratch_types=[
            pltpu.VMEM((vocab,), jnp.float32),
            pltpu.VMEM((2, batch // 2), jnp.int32),
            pltpu.VMEM((batch // 2,), jnp.float32),
            pltpu.SemaphoreType.DMA((2,)),
            pltpu.SemaphoreType.DMA,
        ],
        compiler_params=pltpu.CompilerParams(needs_layout_passes=False),
    )
    def lookup_k(t_hbm, u_hbm, out_hbm, drow_v, idx_v, orow_v, isem, osem):
        wid = lax.axis_index("s") * _NC + lax.axis_index("c")
        half = batch // 2

        def istage(f, h, buf):
            return pltpu.make_async_copy(
                u_hbm.at[f, pl.ds(h * half, half)],
                idx_v.at[buf], isem.at[buf])

        def owrite(f, h):
            return pltpu.make_async_copy(
                orow_v,
                out_hbm.at[f * dim + wid, pl.ds(h * half, half)],
                osem)

        istage(0, 0, 0).start()
        first = True
        for f in range(num_fields):
            pltpu.sync_copy(t_hbm.at[f, wid], drow_v)
            for h in range(2):
                buf = h
                istage(f, h, buf).wait()
                if h == 0:
                    istage(f, 1, 1).start()
                elif f + 1 < num_fields:
                    istage(f + 1, 0, 0).start()
                if not first:
                    owrite(f, h).wait()
                first = False

                def body(j, _, buf=buf):
                    for t in range(16):
                        u = idx_v[buf, pl.ds(j * 256 + t * 16, 16)]
                        orow_v[pl.ds(j * 256 + t * 16, 16)] = (
                            plsc.load_gather(drow_v, [u]))
                    return 0

                lax.fori_loop(0, half // 256, body, 0)
                owrite(f, h).start()
        owrite(num_fields - 1, 1).wait()

    return lookup_k(tables_t, users_t)


def kernel(users, tables):
    num_fields, vocab, dim = tables.shape
    batch = users.shape[0]

    tables_t = jnp.transpose(tables, (0, 2, 1))
    users_t = jnp.transpose(users.astype(jnp.int32), (1, 0))

    out_t = _lookup_call(tables_t, users_t, num_fields, vocab, dim, batch)
    return jnp.transpose(out_t, (1, 0)).reshape(batch, num_fields * dim)


# speedup vs baseline: 1.2524x; 1.2524x over previous
import functools

import jax
import jax.numpy as jnp
from jax import lax
from jax.experimental import pallas as pl
from jax.experimental.pallas import tpu as pltpu
from jax.experimental.pallas import tpu_sc as plsc

_NC = 2
_NS = 16
_NW = _NC * _NS


def _lookup_call(tables_t, users_t, num_fields, vocab, dim, batch):
    mesh = plsc.VectorSubcoreMesh(
        core_axis_name="c", subcore_axis_name="s",
        num_cores=_NC, num_subcores=_NS)

    @functools.partial(
        pl.kernel,
        mesh=mesh,
        out_type=jax.ShapeDtypeStruct((num_fields * dim, batch), jnp.float32),
        scratch_types=[
            pltpu.VMEM((vocab,), jnp.float32),
            pltpu.VMEM((batch // 2,), jnp.int32),
            pltpu.VMEM((batch,), jnp.float32),
            pltpu.SemaphoreType.DMA((2,)),
        ],
        compiler_params=pltpu.CompilerParams(needs_layout_passes=False),
    )
    def lookup_k(t_hbm, u_hbm, out_hbm, drow_v, idx_v, orow_v, osem):
        wid = lax.axis_index("s") * _NC + lax.axis_index("c")
        half = batch // 2

        def owrite(f, h):
            return pltpu.make_async_copy(
                orow_v.at[pl.ds(h * half, half)],
                out_hbm.at[f * dim + wid, pl.ds(h * half, half)],
                osem.at[h])

        for f in range(num_fields):
            pltpu.sync_copy(t_hbm.at[f, wid], drow_v)
            for h in range(2):
                pltpu.sync_copy(u_hbm.at[f, pl.ds(h * half, half)], idx_v)
                if f > 0:
                    owrite(f - 1, h).wait()

                def body(j, _, h=h):
                    for t in range(8):
                        u = idx_v[pl.ds(j * 128 + t * 16, 16)]
                        orow_v[pl.ds(h * half + j * 128 + t * 16, 16)] = (
                            plsc.load_gather(drow_v, [u]))
                    return 0

                lax.fori_loop(0, half // 128, body, 0)
                owrite(f, h).start()
        for h in range(2):
            owrite(num_fields - 1, h).wait()

    return lookup_k(tables_t, users_t)


def kernel(users, tables):
    num_fields, vocab, dim = tables.shape
    batch = users.shape[0]

    tables_t = jnp.transpose(tables, (0, 2, 1))
    users_t = jnp.transpose(users.astype(jnp.int32), (1, 0))

    out_t = _lookup_call(tables_t, users_t, num_fields, vocab, dim, batch)
    return jnp.transpose(out_t, (1, 0)).reshape(batch, num_fields * dim)


# speedup vs baseline: 1.4120x; 1.1275x over previous
import functools

import jax
import jax.numpy as jnp
from jax import lax
from jax.experimental import pallas as pl
from jax.experimental.pallas import tpu as pltpu
from jax.experimental.pallas import tpu_sc as plsc

_NC = 2
_NS = 16
_NW = _NC * _NS
_NBUF = 3


def _lookup_call(tables_t, users_tf, num_fields, vocab, dim, batch):
    mesh = plsc.VectorSubcoreMesh(
        core_axis_name="c", subcore_axis_name="s",
        num_cores=_NC, num_subcores=_NS)

    half = batch // 2
    n_chunks = 2 * num_fields

    @functools.partial(
        pl.kernel,
        mesh=mesh,
        out_type=jax.ShapeDtypeStruct((num_fields * dim, batch), jnp.float32),
        scratch_types=[
            pltpu.VMEM((vocab,), jnp.float32),
            pltpu.VMEM((_NBUF * half,), jnp.float32),
            pltpu.SemaphoreType.DMA((_NBUF,)),
            pltpu.SemaphoreType.DMA((_NBUF,)),
        ],
        compiler_params=pltpu.CompilerParams(needs_layout_passes=False),
    )
    def lookup_k(t_hbm, u_hbm, out_hbm, drow_v, chunk_v, isem, osem):
        wid = lax.axis_index("s") * _NC + lax.axis_index("c")

        def istage(c):
            f, h = divmod(c, 2)
            b = c % _NBUF
            return pltpu.make_async_copy(
                u_hbm.at[f, pl.ds(h * half, half)],
                chunk_v.at[pl.ds(b * half, half)], isem.at[b])

        def owrite(c):
            f, h = divmod(c, 2)
            b = c % _NBUF
            return pltpu.make_async_copy(
                chunk_v.at[pl.ds(b * half, half)],
                out_hbm.at[f * dim + wid, pl.ds(h * half, half)],
                osem.at[b])

        istage(0).start()
        for c in range(n_chunks):
            f, h = divmod(c, 2)
            b = c % _NBUF
            if h == 0:
                pltpu.sync_copy(t_hbm.at[f, wid], drow_v)
            istage(c).wait()
            if c + 1 < n_chunks:
                if c + 1 >= _NBUF:
                    owrite(c + 1 - _NBUF).wait()
                istage(c + 1).start()

            def body(j, _, b=b):
                for t in range(8):
                    u_raw = chunk_v[pl.ds(b * half + j * 128 + t * 16, 16)]
                    u = plsc.bitcast(u_raw, jnp.int32)
                    chunk_v[pl.ds(b * half + j * 128 + t * 16, 16)] = (
                        plsc.load_gather(drow_v, [u]))
                return 0

            lax.fori_loop(0, half // 128, body, 0)
            owrite(c).start()
        for c in range(n_chunks - _NBUF + 1, n_chunks):
            owrite(c).wait()

    return lookup_k(tables_t, users_tf)


def kernel(users, tables):
    num_fields, vocab, dim = tables.shape
    batch = users.shape[0]

    tables_t = jnp.transpose(tables, (0, 2, 1))
    users_t = jnp.transpose(users.astype(jnp.int32), (1, 0))
    users_tf = lax.bitcast_convert_type(users_t, jnp.float32)

    out_t = _lookup_call(tables_t, users_tf, num_fields, vocab, dim, batch)
    return jnp.transpose(out_t, (1, 0)).reshape(batch, num_fields * dim)
